# baseline (device time: 15832 ns/iter reference)
import os as _os

import jax
import jax.numpy as jnp
from jax import lax
from jax.experimental import pallas as pl
from jax.experimental.pallas import tpu as pltpu

N_DEV = 4
B, SQ, DM = 2, 256, 512
HQ, DH = 4, 64
SKV = 256
BLK = 64

_DIAG_NO_COMM = bool(_os.environ.get("DIAG_NO_COMM"))


def _compute(x_ref, wq_ref, wo_ref, out_ref, commk, commv,
             wait_k=None, wait_v=None):
    i_blk = lax.broadcasted_iota(jnp.int32, (SQ, SKV), 0) // BLK
    j_blk = lax.broadcasted_iota(jnp.int32, (SQ, SKV), 1) // BLK
    mask = j_blk <= i_blk
    qs = [(jnp.dot(x_ref[b], wq_ref[...],
                   preferred_element_type=jnp.float32) * 0.125)
          for b in range(B)]

    if wait_k is not None:
        wait_k()

    ws = []
    for b in range(B):
        kb = commk[b]
        for h in range(HQ):
            sl = slice(h * DH, (h + 1) * DH)
            scores = lax.dot_general(
                qs[b][:, sl].astype(jnp.bfloat16), kb[:, sl],
                (((1,), (1,)), ((), ())),
                preferred_element_type=jnp.float32)
            w = jnp.where(mask, jnp.exp(scores), 0.0)
            w = w / jnp.sum(w, axis=1, keepdims=True)
            ws.append(w.astype(jnp.bfloat16))

    if wait_v is not None:
        wait_v()

    for b in range(B):
        vb = commv[b]
        ctx = jnp.concatenate(
            [jnp.dot(ws[b * HQ + h], vb[:, h * DH:(h + 1) * DH],
                     preferred_element_type=jnp.float32)
             for h in range(HQ)], axis=1).astype(jnp.bfloat16)
        out_ref[b] = jnp.dot(ctx, wo_ref[...],
                             preferred_element_type=jnp.float32)


def _body(x_ref, wq_ref, k_ref, v_ref, wo_ref, out_ref,
          commk, commv, send_sems, recvk_sem, recvv_sem):
    my = lax.axis_index("i")

    if _DIAG_NO_COMM:
        commk[...] = k_ref[...]
        commv[...] = v_ref[...]
        _compute(x_ref, wq_ref, wo_ref, out_ref, commk, commv)
        return

    left = (my + N_DEV - 1) % N_DEV
    right = (my + 1) % N_DEV
    barrier = pltpu.get_barrier_semaphore()
    for nbr in (left, right):
        pl.semaphore_signal(barrier, inc=1, device_id=(nbr,),
                            device_id_type=pl.DeviceIdType.MESH)
    pl.semaphore_wait(barrier, 2)

    def _send(src, dst, sem_idx, rsem, dev):
        return pltpu.make_async_remote_copy(
            src_ref=src, dst_ref=dst, send_sem=send_sems.at[sem_idx],
            recv_sem=rsem, device_id=(dev,),
            device_id_type=pl.DeviceIdType.MESH)

    def _recv(dst, rsem):
        return _send(dst, dst, 0, rsem, 0)

    @pl.when(my == 0)
    def _():
        _send(k_ref, commk, 0, recvk_sem, 1).start()
        _send(v_ref, commv, 1, recvv_sem, 3).start()
        _send(v_ref, commv, 2, recvv_sem, 1).start()
        _send(k_ref, commk, 3, recvk_sem, 3).start()
        commk[...] = k_ref[...]
        commv[...] = v_ref[...]

    def wait_k():
        @pl.when(my == 1)
        def _():
            _recv(commk, recvk_sem).wait_recv()
            _send(commk, commk, 0, recvk_sem, 2).start()

        @pl.when(my == 3)
        def _():
            _recv(commv, recvv_sem).wait_recv()
            _send(commv, commv, 0, recvv_sem, 2).start()
            _recv(commk, recvk_sem).wait_recv()

        @pl.when(my == 2)
        def _():
            _recv(commk, recvk_sem).wait_recv()

    def wait_v():
        @pl.when((my == 1) | (my == 2))
        def _():
            _recv(commv, recvv_sem).wait_recv()

    _compute(x_ref, wq_ref, wo_ref, out_ref, commk, commv, wait_k, wait_v)

    @pl.when(my == 0)
    def _():
        for idx in range(4):
            _send(k_ref, commk, idx, recvk_sem, 1).wait_send()

    @pl.when((my == 1) | (my == 3))
    def _():
        _send(commk, commk, 0, recvk_sem, 2).wait_send()


def kernel(x, Wq, K_ext, V_ext, Wo):
    bf = jnp.bfloat16
    k2 = K_ext.astype(bf).reshape(B, SKV, HQ * DH)
    v2 = V_ext.astype(bf).reshape(B, SKV, HQ * DH)
    return pl.pallas_call(
        _body,
        out_shape=jax.ShapeDtypeStruct((B, SQ, DM), jnp.float32),
        in_specs=[pl.BlockSpec(memory_space=pltpu.VMEM)] * 5,
        out_specs=pl.BlockSpec(memory_space=pltpu.VMEM),
        scratch_shapes=[
            pltpu.VMEM((B, SKV, HQ * DH), bf),
            pltpu.VMEM((B, SKV, HQ * DH), bf),
            pltpu.SemaphoreType.DMA((4,)),
            pltpu.SemaphoreType.DMA,
            pltpu.SemaphoreType.DMA,
        ],
        compiler_params=(None if _DIAG_NO_COMM
                         else pltpu.CompilerParams(collective_id=0)),
    )(x.astype(bf), Wq.astype(bf), k2, v2, Wo.astype(bf))


# device time: 6540 ns/iter; 2.4208x vs baseline; 2.4208x over previous
import os as _os

import jax
import jax.numpy as jnp
from jax import lax
from jax.experimental import pallas as pl
from jax.experimental.pallas import tpu as pltpu

N_DEV = 4
B, SQ, DM = 2, 256, 512
HQ, DH = 4, 64
SKV = 256
BLK = 64

_DIAG_NO_COMM = bool(_os.environ.get("DIAG_NO_COMM"))


def _compute(x_ref, wq_ref, wo_ref, out_ref, commk, commv,
             wait_k=None, wait_v=None):
    i_blk = lax.broadcasted_iota(jnp.int32, (SQ, SKV), 0) // BLK
    j_blk = lax.broadcasted_iota(jnp.int32, (SQ, SKV), 1) // BLK
    mask = j_blk <= i_blk
    qs = [(jnp.dot(x_ref[b], wq_ref[...],
                   preferred_element_type=jnp.float32) * 0.125)
          for b in range(B)]

    ws = []
    for b in range(B):
        if wait_k is not None:
            wait_k(b)
        kb = commk[b]
        for h in range(HQ):
            sl = slice(h * DH, (h + 1) * DH)
            scores = lax.dot_general(
                qs[b][:, sl].astype(jnp.bfloat16), kb[:, sl],
                (((1,), (1,)), ((), ())),
                preferred_element_type=jnp.float32)
            w = jnp.where(mask, jnp.exp(scores), 0.0)
            w = w / jnp.sum(w, axis=1, keepdims=True)
            ws.append(w.astype(jnp.bfloat16))

    for b in range(B):
        if wait_v is not None:
            wait_v(b)
        vb = commv[b]
        ctx = jnp.concatenate(
            [jnp.dot(ws[b * HQ + h], vb[:, h * DH:(h + 1) * DH],
                     preferred_element_type=jnp.float32)
             for h in range(HQ)], axis=1).astype(jnp.bfloat16)
        out_ref[b] = jnp.dot(ctx, wo_ref[...],
                             preferred_element_type=jnp.float32)


def _body(x_ref, wq_ref, k_ref, v_ref, wo_ref, out_ref,
          commk, commv, send_sems, recvk_sems, recvv_sems):
    my = lax.axis_index("i")

    if _DIAG_NO_COMM:
        commk[...] = k_ref[...]
        commv[...] = v_ref[...]
        _compute(x_ref, wq_ref, wo_ref, out_ref, commk, commv)
        return

    left = (my + N_DEV - 1) % N_DEV
    right = (my + 1) % N_DEV
    barrier = pltpu.get_barrier_semaphore()
    for nbr in (left, right):
        pl.semaphore_signal(barrier, inc=1, device_id=(nbr,),
                            device_id_type=pl.DeviceIdType.MESH)
    pl.semaphore_wait(barrier, 2)

    def _chunk_copy(src, dst, b, send_idx, rsem, dev):
        return pltpu.make_async_remote_copy(
            src_ref=src.at[b], dst_ref=dst.at[b],
            send_sem=send_sems.at[send_idx], recv_sem=rsem,
            device_id=(dev,), device_id_type=pl.DeviceIdType.MESH)

    def _recv(dst, b, rsem):
        return _chunk_copy(dst, dst, b, 0, rsem, 0)

    @pl.when(my == 0)
    def _():
        _chunk_copy(k_ref, commk, 0, 0, recvk_sems.at[0], 1).start()
        _chunk_copy(k_ref, commk, 1, 1, recvk_sems.at[1], 1).start()
        _chunk_copy(v_ref, commv, 0, 2, recvv_sems.at[0], 3).start()
        _chunk_copy(v_ref, commv, 1, 3, recvv_sems.at[1], 3).start()
        _chunk_copy(v_ref, commv, 0, 4, recvv_sems.at[0], 1).start()
        _chunk_copy(v_ref, commv, 1, 5, recvv_sems.at[1], 1).start()
        _chunk_copy(k_ref, commk, 0, 6, recvk_sems.at[0], 3).start()
        _chunk_copy(k_ref, commk, 1, 7, recvk_sems.at[1], 3).start()
        commk[...] = k_ref[...]
        commv[...] = v_ref[...]

    def wait_k(b):
        @pl.when(my == 1)
        def _():
            _recv(commk, b, recvk_sems.at[b]).wait_recv()
            _chunk_copy(commk, commk, b, b, recvk_sems.at[b], 2).start()

        @pl.when(my == 3)
        def _():
            _recv(commv, b, recvv_sems.at[b]).wait_recv()
            _chunk_copy(commv, commv, b, b, recvv_sems.at[b], 2).start()
            _recv(commk, b, recvk_sems.at[b]).wait_recv()

        @pl.when(my == 2)
        def _():
            _recv(commk, b, recvk_sems.at[b]).wait_recv()

    def wait_v(b):
        @pl.when((my == 1) | (my == 2))
        def _():
            _recv(commv, b, recvv_sems.at[b]).wait_recv()

    _compute(x_ref, wq_ref, wo_ref, out_ref, commk, commv, wait_k, wait_v)

    @pl.when(my == 0)
    def _():
        for idx in range(8):
            _chunk_copy(k_ref, commk, 0, idx, recvk_sems.at[0], 1).wait_send()

    @pl.when((my == 1) | (my == 3))
    def _():
        for idx in range(2):
            _chunk_copy(commk, commk, 0, idx, recvk_sems.at[0], 2).wait_send()


def kernel(x, Wq, K_ext, V_ext, Wo):
    bf = jnp.bfloat16
    k2 = K_ext.astype(bf).reshape(B, SKV, HQ * DH)
    v2 = V_ext.astype(bf).reshape(B, SKV, HQ * DH)
    return pl.pallas_call(
        _body,
        out_shape=jax.ShapeDtypeStruct((B, SQ, DM), jnp.float32),
        in_specs=[pl.BlockSpec(memory_space=pltpu.VMEM)] * 5,
        out_specs=pl.BlockSpec(memory_space=pltpu.VMEM),
        scratch_shapes=[
            pltpu.VMEM((B, SKV, HQ * DH), bf),
            pltpu.VMEM((B, SKV, HQ * DH), bf),
            pltpu.SemaphoreType.DMA((8,)),
            pltpu.SemaphoreType.DMA((2,)),
            pltpu.SemaphoreType.DMA((2,)),
        ],
        compiler_params=(None if _DIAG_NO_COMM
                         else pltpu.CompilerParams(collective_id=0)),
    )(x.astype(bf), Wq.astype(bf), k2, v2, Wo.astype(bf))
